# baseline (speedup 1.0000x reference)
import functools

import jax
import jax.numpy as jnp
from jax import lax
from jax.experimental import pallas as pl
from jax.experimental.pallas import tpu as pltpu
from jax.experimental.pallas import tpu_sc as plsc

NU, NM, NE, NL, H, MF = 50000, 10000, 500000, 20000, 128, 404
NU_P, NM_P, NE_P, NL_P = 50176, 10240, 512000, 20480
N_TILES = 16
f32 = jnp.float32
i32 = jnp.int32


@functools.cache
def _mesh():
    return dict(
        mesh=plsc.VectorSubcoreMesh(core_axis_name="core",
                                    subcore_axis_name="subcore"),
        compiler_params=pltpu.CompilerParams(use_tc_tiling_on_sc=False))


def _seg_pass(table_ref, gidx_hbm, sidx_hbm, acc_sh, ig_v, is_v, buf_v,
              semg, sems, semi, base_row, n_groups, K, B):
    def idx_prefetch(g, par):
        r = base_row + g * K
        pltpu.async_copy(gidx_hbm.at[pl.ds(r, K), :], ig_v.at[par], semi)
        pltpu.async_copy(sidx_hbm.at[pl.ds(r, K), :], is_v.at[par], semi)

    def idx_wait(par):
        pltpu.make_async_copy(gidx_hbm.at[pl.ds(base_row, K), :],
                              ig_v.at[par], semi).wait()
        pltpu.make_async_copy(sidx_hbm.at[pl.ds(base_row, K), :],
                              is_v.at[par], semi).wait()

    def wait_scatters(par):
        for i in range(K):
            pltpu.make_async_copy(buf_v.at[par].at[pl.ds(i * B, B), :],
                                  acc_sh.at[is_v.at[par, i]], sems).wait()

    def group(g, par, first, last):
        idx_wait(par)
        ds = [pltpu.async_copy(table_ref.at[ig_v.at[par, i]],
                               buf_v.at[par].at[pl.ds(i * B, B), :], semg)
              for i in range(K)]
        for d in ds:
            d.wait()
        if not first:
            wait_scatters(1 - par)
        if not last:
            idx_prefetch(g + 1, 1 - par)
        for i in range(K):
            pltpu.async_copy(buf_v.at[par].at[pl.ds(i * B, B), :],
                             acc_sh.at[is_v.at[par, i]], sems, add=True)

    assert n_groups % 2 == 0
    idx_prefetch(0, 0)
    group(0, 0, True, False)

    def body(t, carry):
        group(2 * t + 1, 1, False, False)
        group(2 * t + 2, 0, False, False)
        return carry
    lax.fori_loop(0, (n_groups - 2) // 2, body, 0)
    group(n_groups - 1, 1, False, True)
    wait_scatters(1)


def _sc_counts_body(rs2d_hbm, rd2d_hbm, ones_hbm, zu_hbm, zm_hbm,
                    cntu_hbm, cntm_hbm,
                    acc_sh, ix_v, ones_v, sems, semi):
    c = lax.axis_index("core")
    s = lax.axis_index("subcore")
    K, B = 5, 128
    pltpu.sync_copy(ones_hbm, ones_v)

    def side(idx_hbm, out_hbm, z_hbm, per):
        pltpu.sync_copy(z_hbm, acc_sh.at[pl.ds(s * per, per), :])
        plsc.subcore_barrier()

        def prefetch(g, par):
            pltpu.async_copy(idx_hbm.at[pl.ds(s * 250 + g * K, K), :],
                             ix_v.at[par], semi)

        def idx_wait(par):
            pltpu.make_async_copy(idx_hbm.at[pl.ds(s * 250, K), :],
                                  ix_v.at[par], semi).wait()

        def wait_sc(par):
            for i in range(K):
                pltpu.make_async_copy(ones_v, acc_sh.at[ix_v.at[par, i]],
                                      sems).wait()

        def group(g, par, first, last):
            idx_wait(par)
            if not first:
                wait_sc(1 - par)
            if not last:
                prefetch(g + 1, 1 - par)
            for i in range(K):
                pltpu.async_copy(ones_v, acc_sh.at[ix_v.at[par, i]], sems,
                                 add=True)

        prefetch(0, 0)
        group(0, 0, True, False)

        def body(t, carry):
            group(2 * t + 1, 1, False, False)
            group(2 * t + 2, 0, False, False)
            return carry
        lax.fori_loop(0, 24, body, 0)
        group(49, 1, False, True)
        wait_sc(1)
        plsc.subcore_barrier()
        pltpu.sync_copy(acc_sh.at[pl.ds(s * per, per), :],
                        out_hbm.at[pl.ds(s * per, per), :])

    @pl.when(c == 0)
    def _():
        side(rs2d_hbm, cntu_hbm, zu_hbm, NU_P // N_TILES)

    @pl.when(c == 1)
    def _():
        side(rd2d_hbm, cntm_hbm, zm_hbm, NM_P // N_TILES)


def _sc_counts(rs2d, rd2d, ones16, zu, zm):
    return pl.kernel(
        _sc_counts_body,
        out_type=(jax.ShapeDtypeStruct((NU_P, 16), f32),
                  jax.ShapeDtypeStruct((NM_P, 16), f32)),
        scratch_types=[
            pltpu.VMEM_SHARED((NU_P, 16), f32),
            pltpu.VMEM((2, 5, 128), i32),
            pltpu.VMEM((128, 16), f32),
            pltpu.SemaphoreType.DMA,
            pltpu.SemaphoreType.DMA,
        ],
        **_mesh(),
    )(rs2d, rd2d, ones16, zu, zm)


def _sc_pass1_body(pqc_hbm, rs2d_hbm, rd2d_hbm, z_hbm, out_hbm,
                   acc_sh, ig_v, is_v, buf_v, semg, sems, semi):
    c = lax.axis_index("core")
    s = lax.axis_index("subcore")
    per = NU_P // N_TILES
    for j in range(4):
        chunk = c * 4 + j
        pltpu.sync_copy(z_hbm, acc_sh.at[pl.ds(s * per, per), :])
        plsc.subcore_barrier()
        _seg_pass(pqc_hbm.at[chunk], rd2d_hbm, rs2d_hbm, acc_sh,
                  ig_v, is_v, buf_v, semg, sems, semi,
                  base_row=s * 800, n_groups=80, K=10, B=40)
        plsc.subcore_barrier()
        pltpu.sync_copy(acc_sh.at[pl.ds(s * per, per), :],
                        out_hbm.at[chunk].at[pl.ds(s * per, per), :])
        plsc.subcore_barrier()


def _sc_pass1(pqc, rs2d, rd2d, zer):
    return pl.kernel(
        _sc_pass1_body,
        out_type=jax.ShapeDtypeStruct((8, NU_P, 32), f32),
        scratch_types=[
            pltpu.VMEM_SHARED((NU_P, 32), f32),
            pltpu.VMEM((2, 10, 40), i32),
            pltpu.VMEM((2, 10, 40), i32),
            pltpu.VMEM((2, 400, 32), f32),
            pltpu.SemaphoreType.DMA,
            pltpu.SemaphoreType.DMA,
            pltpu.SemaphoreType.DMA,
        ],
        **_mesh(),
    )(pqc, rs2d, rd2d, zer)


def _sc_pass2_body(r_hbm, rs2d_hbm, rd2d_hbm, z_hbm, out_hbm,
                   acc_sh, ig_v, is_v, buf_v, semg, sems, semi):
    c = lax.axis_index("core")
    s = lax.axis_index("subcore")
    per = NM_P // N_TILES
    w = c * N_TILES + s
    pltpu.sync_copy(z_hbm, acc_sh.at[pl.ds(s * per, per), :])
    plsc.subcore_barrier()
    _seg_pass(r_hbm, rs2d_hbm, rd2d_hbm, acc_sh, ig_v, is_v, buf_v,
              semg, sems, semi, base_row=w * 1000, n_groups=100, K=10, B=16)
    plsc.subcore_barrier()
    pltpu.sync_copy(acc_sh.at[pl.ds(s * per, per), :],
                    out_hbm.at[c].at[pl.ds(s * per, per), :])


def _sc_pass2(r_full, rs2d32, rd2d32, zer):
    return pl.kernel(
        _sc_pass2_body,
        out_type=jax.ShapeDtypeStruct((2, NM_P, H), f32),
        scratch_types=[
            pltpu.VMEM_SHARED((NM_P, H), f32),
            pltpu.VMEM((2, 10, 16), i32),
            pltpu.VMEM((2, 10, 16), i32),
            pltpu.VMEM((2, 160, H), f32),
            pltpu.SemaphoreType.DMA,
            pltpu.SemaphoreType.DMA,
            pltpu.SemaphoreType.DMA,
        ],
        **_mesh(),
    )(r_full, rs2d32, rd2d32, zer)


def _sc_gather_body(u2_hbm, m2_hbm, ls2d_hbm, ld2d_hbm, u_hbm, m_hbm,
                    iu_v, im_v, bu_v, bm_v, semg, semw):
    c = lax.axis_index("core")
    s = lax.axis_index("subcore")
    B = 128
    w = s * 2 + c
    nb = NL_P // (2 * N_TILES * B)

    def wait_wb(par, base):
        pltpu.make_async_copy(bu_v.at[par], u_hbm.at[pl.ds(base, B), :],
                              semw).wait()
        pltpu.make_async_copy(bm_v.at[par], m_hbm.at[pl.ds(base, B), :],
                              semw).wait()

    for b in range(nb):
        par = b % 2
        base = (w * nb + b) * B
        pltpu.sync_copy(ls2d_hbm.at[pl.ds(w * nb + b, 1), :], iu_v.at[par])
        pltpu.sync_copy(ld2d_hbm.at[pl.ds(w * nb + b, 1), :], im_v.at[par])
        if b >= 2:
            wait_wb(par, base)
        d1 = pltpu.async_copy(u2_hbm.at[iu_v.at[par, 0]], bu_v.at[par], semg)
        d2 = pltpu.async_copy(m2_hbm.at[im_v.at[par, 0]], bm_v.at[par], semg)
        d1.wait()
        d2.wait()
        pltpu.async_copy(bu_v.at[par], u_hbm.at[pl.ds(base, B), :], semw)
        pltpu.async_copy(bm_v.at[par], m_hbm.at[pl.ds(base, B), :], semw)
    wait_wb((nb - 2) % 2, 0)
    wait_wb((nb - 1) % 2, 0)


def _sc_gather(u2, m2, ls2d, ld2d):
    return pl.kernel(
        _sc_gather_body,
        out_type=(jax.ShapeDtypeStruct((NL_P, H), f32),
                  jax.ShapeDtypeStruct((NL_P, H), f32)),
        scratch_types=[
            pltpu.VMEM((2, 1, 128), i32),
            pltpu.VMEM((2, 1, 128), i32),
            pltpu.VMEM((2, 128, H), f32),
            pltpu.VMEM((2, 128, H), f32),
            pltpu.SemaphoreType.DMA,
            pltpu.SemaphoreType.DMA,
        ],
        **_mesh(),
    )(u2, m2, ls2d, ld2d)


def _tc_movies_body(x_ref, wm_ref, bm_ref, wr1_ref, br1_ref, wv1_ref,
                    wv2_ref, m1_ref, pqc_ref):
    x = x_ref[...]
    m0 = jnp.dot(x, wm_ref[...], preferred_element_type=f32) + bm_ref[...]
    m1 = jnp.maximum(
        jnp.dot(m0, wr1_ref[...], preferred_element_type=f32) + br1_ref[...],
        0.0)
    p = jnp.dot(m0, wv1_ref[...], preferred_element_type=f32)
    q = jnp.dot(m1, wv2_ref[...], preferred_element_type=f32)
    m1_ref[...] = m1
    pq = jnp.concatenate([p, q], axis=1)
    blk = pq.shape[0]
    pqc_ref[...] = jnp.transpose(pq.reshape(blk, 8, 32), (1, 0, 2))


def _tc_movies(x_p, wm, bm, wr1, br1, wv1, wv2):
    blk = 400
    g = NM // blk
    return pl.pallas_call(
        _tc_movies_body,
        grid=(g,),
        in_specs=[
            pl.BlockSpec((blk, MF), lambda i: (i, 0)),
            pl.BlockSpec((MF, H), lambda i: (0, 0)),
            pl.BlockSpec((1, H), lambda i: (0, 0)),
            pl.BlockSpec((H, H), lambda i: (0, 0)),
            pl.BlockSpec((1, H), lambda i: (0, 0)),
            pl.BlockSpec((H, H), lambda i: (0, 0)),
            pl.BlockSpec((H, H), lambda i: (0, 0)),
        ],
        out_specs=[
            pl.BlockSpec((blk, H), lambda i: (i, 0)),
            pl.BlockSpec((8, blk, 32), lambda i: (0, i, 0)),
        ],
        out_shape=[
            jax.ShapeDtypeStruct((NM_P, H), f32),
            jax.ShapeDtypeStruct((8, NM_P, 32), f32),
        ],
    )(x_p, wm, bm, wr1, br1, wv1, wv2)


def _tc_user_body(spq_ref, cnt_ref, wlr2_ref, wrv2_ref, bv1_ref, bv2_ref,
                  r_ref, u2_ref):
    spq = spq_ref[...]
    inv = 1.0 / jnp.maximum(cnt_ref[...], 1.0)
    b1 = bv1_ref[...]
    b2 = bv2_ref[...]
    u1 = jnp.concatenate(
        [jnp.maximum(spq[j] * inv + b1[:, 32 * j:32 * j + 32], 0.0)
         for j in range(4)], axis=1)
    r_ref[...] = jnp.dot(u1, wlr2_ref[...], preferred_element_type=f32)
    u2_ref[...] = jnp.concatenate(
        [spq[4 + j] * inv + b2[:, 32 * j:32 * j + 32] for j in range(4)],
        axis=1) + jnp.dot(u1, wrv2_ref[...], preferred_element_type=f32)


def _tc_user(spq, cnt_u, wlr2, wrv2, bv1, bv2):
    blk = 512
    g = NU_P // blk
    return pl.pallas_call(
        _tc_user_body,
        grid=(g,),
        in_specs=[
            pl.BlockSpec((8, blk, 32), lambda i: (0, i, 0)),
            pl.BlockSpec((blk, 1), lambda i: (i, 0)),
            pl.BlockSpec((H, H), lambda i: (0, 0)),
            pl.BlockSpec((H, H), lambda i: (0, 0)),
            pl.BlockSpec((1, H), lambda i: (0, 0)),
            pl.BlockSpec((1, H), lambda i: (0, 0)),
        ],
        out_specs=[
            pl.BlockSpec((blk, H), lambda i: (i, 0)),
            pl.BlockSpec((blk, H), lambda i: (i, 0)),
        ],
        out_shape=[
            jax.ShapeDtypeStruct((NU_P, H), f32),
            jax.ShapeDtypeStruct((NU_P, H), f32),
        ],
    )(spq, cnt_u, wlr2, wrv2, bv1, bv2)


def _tc_movie2_body(sr_ref, cnt_ref, m1_ref, wr2_ref, br2_ref, m2_ref):
    sr = sr_ref[...]
    inv = 1.0 / jnp.maximum(cnt_ref[...], 1.0)
    m2_ref[...] = ((sr[0] + sr[1]) * inv + br2_ref[...] +
                   jnp.dot(m1_ref[...], wr2_ref[...],
                           preferred_element_type=f32))


def _tc_movie2(sr, cnt_m, m1, wr2, br2):
    blk = 512
    g = NM_P // blk
    return pl.pallas_call(
        _tc_movie2_body,
        grid=(g,),
        in_specs=[
            pl.BlockSpec((2, blk, H), lambda i: (0, i, 0)),
            pl.BlockSpec((blk, 1), lambda i: (i, 0)),
            pl.BlockSpec((blk, H), lambda i: (i, 0)),
            pl.BlockSpec((H, H), lambda i: (0, 0)),
            pl.BlockSpec((1, H), lambda i: (0, 0)),
        ],
        out_specs=pl.BlockSpec((blk, H), lambda i: (i, 0)),
        out_shape=jax.ShapeDtypeStruct((NM_P, H), f32),
    )(sr, cnt_m, m1, wr2, br2)


def _tc_dot_body(u_ref, m_ref, o_ref):
    o_ref[...] = jnp.sum(u_ref[...] * m_ref[...], axis=1, keepdims=True)


def _tc_dot(u, m):
    blk = 512
    g = NL_P // blk
    return pl.pallas_call(
        _tc_dot_body,
        grid=(g,),
        in_specs=[
            pl.BlockSpec((blk, H), lambda i: (i, 0)),
            pl.BlockSpec((blk, H), lambda i: (i, 0)),
        ],
        out_specs=pl.BlockSpec((blk, 1), lambda i: (i, 0)),
        out_shape=jax.ShapeDtypeStruct((NL_P, 1), f32),
    )(u, m)


def kernel(movie_x, W_movie, b_movie, Wl_r1, bl_r1, Wr_r1, Wl_v1, bl_v1,
           Wr_v1, Wl_r2, bl_r2, Wr_r2, Wl_v2, bl_v2, Wr_v2,
           rates_src, rates_dst, label_src, label_dst):
    pad_u = NU + jnp.arange(NE_P - NE, dtype=i32) % (NU_P - NU)
    pad_m = NM + jnp.arange(NE_P - NE, dtype=i32) % (NM_P - NM)
    rs_p = jnp.concatenate([rates_src.astype(i32), pad_u])
    rd_p = jnp.concatenate([rates_dst.astype(i32), pad_m])
    rs2d = rs_p.reshape(NE_P // 128, 128)
    rd2d = rd_p.reshape(NE_P // 128, 128)
    rs2d64 = rs_p.reshape(NE_P // 64, 64)
    rd2d64 = rd_p.reshape(NE_P // 64, 64)
    rs2d32 = rs_p.reshape(NE_P // 32, 32)
    rd2d32 = rd_p.reshape(NE_P // 32, 32)
    rs2d40 = rs_p.reshape(NE_P // 40, 40)
    rd2d40 = rd_p.reshape(NE_P // 40, 40)
    rs2d16 = rs_p.reshape(NE_P // 16, 16)
    rd2d16 = rd_p.reshape(NE_P // 16, 16)
    ls2d = jnp.concatenate([label_src.astype(i32),
                            jnp.zeros((NL_P - NL,), i32)]).reshape(
                                NL_P // 128, 128)
    ld2d = jnp.concatenate([label_dst.astype(i32),
                            jnp.zeros((NL_P - NL,), i32)]).reshape(
                                NL_P // 128, 128)
    bm = b_movie.reshape(1, H)
    br1 = bl_r1.reshape(1, H)
    bv1 = bl_v1.reshape(1, H)
    bv2 = bl_v2.reshape(1, H)
    br2 = bl_r2.reshape(1, H)

    ones16 = jnp.ones((128, 16), f32)
    zu16 = jnp.zeros((NU_P // N_TILES, 16), f32)
    zm16 = jnp.zeros((NM_P // N_TILES, 16), f32)
    z1 = jnp.zeros((NU_P // N_TILES, 32), f32)
    z2 = jnp.zeros((NM_P // N_TILES, H), f32)

    movie1, pqc = _tc_movies(movie_x, W_movie, bm, Wr_r1, br1, Wl_v1, Wl_v2)

    cu16, cm16 = _sc_counts(rs2d, rd2d, ones16, zu16, zm16)
    cnt_u = cu16[:, :1]
    cnt_m = cm16[:, :1]

    spq = _sc_pass1(pqc, rs2d40, rd2d40, z1)
    r_full, user2 = _tc_user(spq, cnt_u, Wl_r2, Wr_v2, bv1, bv2)
    sr = _sc_pass2(r_full, rs2d16, rd2d16, z2)
    movie2 = _tc_movie2(sr, cnt_m, movie1, Wr_r2, br2)
    u_rows, m_rows = _sc_gather(user2, movie2, ls2d, ld2d)
    out = _tc_dot(u_rows, m_rows)
    return out.reshape(NL_P)[:NL]


# speedup vs baseline: 1.0021x; 1.0021x over previous
import functools

import jax
import jax.numpy as jnp
from jax import lax
from jax.experimental import pallas as pl
from jax.experimental.pallas import tpu as pltpu
from jax.experimental.pallas import tpu_sc as plsc

NU, NM, NE, NL, H, MF = 50000, 10000, 500000, 20000, 128, 404
NU_P, NM_P, NE_P, NL_P = 50176, 10240, 512000, 20480
N_TILES = 16
f32 = jnp.float32
i32 = jnp.int32


@functools.cache
def _mesh():
    return dict(
        mesh=plsc.VectorSubcoreMesh(core_axis_name="core",
                                    subcore_axis_name="subcore"),
        compiler_params=pltpu.CompilerParams(use_tc_tiling_on_sc=False))


def _seg_pass(table_ref, gidx_hbm, sidx_hbm, acc_sh, ig_v, is_v, buf_v,
              semg, sems, semi, base_row, n_groups, K, B):
    def idx_prefetch(g, par):
        r = base_row + g * K
        pltpu.async_copy(gidx_hbm.at[pl.ds(r, K), :], ig_v.at[par], semi)
        pltpu.async_copy(sidx_hbm.at[pl.ds(r, K), :], is_v.at[par], semi)

    def idx_wait(par):
        pltpu.make_async_copy(gidx_hbm.at[pl.ds(base_row, K), :],
                              ig_v.at[par], semi).wait()
        pltpu.make_async_copy(sidx_hbm.at[pl.ds(base_row, K), :],
                              is_v.at[par], semi).wait()

    def wait_scatters(par):
        for i in range(K):
            pltpu.make_async_copy(buf_v.at[par].at[pl.ds(i * B, B), :],
                                  acc_sh.at[is_v.at[par, i]], sems).wait()

    def group(g, par, first, last):
        idx_wait(par)
        ds = [pltpu.async_copy(table_ref.at[ig_v.at[par, i]],
                               buf_v.at[par].at[pl.ds(i * B, B), :], semg)
              for i in range(K)]
        for d in ds:
            d.wait()
        if not first:
            wait_scatters(1 - par)
        if not last:
            idx_prefetch(g + 1, 1 - par)
        for i in range(K):
            pltpu.async_copy(buf_v.at[par].at[pl.ds(i * B, B), :],
                             acc_sh.at[is_v.at[par, i]], sems, add=True)

    assert n_groups % 2 == 0
    idx_prefetch(0, 0)
    group(0, 0, True, False)

    def body(t, carry):
        group(2 * t + 1, 1, False, False)
        group(2 * t + 2, 0, False, False)
        return carry
    lax.fori_loop(0, (n_groups - 2) // 2, body, 0)
    group(n_groups - 1, 1, False, True)
    wait_scatters(1)


def _sc_counts_body(rs2d_hbm, rd2d_hbm, ones_hbm, zu_hbm, zm_hbm,
                    cntu_hbm, cntm_hbm,
                    acc_sh, ix_v, ones_v, sems, semi):
    c = lax.axis_index("core")
    s = lax.axis_index("subcore")
    K, B = 5, 128
    pltpu.sync_copy(ones_hbm, ones_v)

    def side(idx_hbm, out_hbm, z_hbm, per):
        pltpu.sync_copy(z_hbm, acc_sh.at[pl.ds(s * per, per), :])
        plsc.subcore_barrier()

        def prefetch(g, par):
            pltpu.async_copy(idx_hbm.at[pl.ds(s * 250 + g * K, K), :],
                             ix_v.at[par], semi)

        def idx_wait(par):
            pltpu.make_async_copy(idx_hbm.at[pl.ds(s * 250, K), :],
                                  ix_v.at[par], semi).wait()

        def wait_sc(par):
            for i in range(K):
                pltpu.make_async_copy(ones_v, acc_sh.at[ix_v.at[par, i]],
                                      sems).wait()

        def group(g, par, first, last):
            idx_wait(par)
            if not first:
                wait_sc(1 - par)
            if not last:
                prefetch(g + 1, 1 - par)
            for i in range(K):
                pltpu.async_copy(ones_v, acc_sh.at[ix_v.at[par, i]], sems,
                                 add=True)

        prefetch(0, 0)
        group(0, 0, True, False)

        def body(t, carry):
            group(2 * t + 1, 1, False, False)
            group(2 * t + 2, 0, False, False)
            return carry
        lax.fori_loop(0, 24, body, 0)
        group(49, 1, False, True)
        wait_sc(1)
        plsc.subcore_barrier()
        pltpu.sync_copy(acc_sh.at[pl.ds(s * per, per), :],
                        out_hbm.at[pl.ds(s * per, per), :])

    @pl.when(c == 0)
    def _():
        side(rs2d_hbm, cntu_hbm, zu_hbm, NU_P // N_TILES)

    @pl.when(c == 1)
    def _():
        side(rd2d_hbm, cntm_hbm, zm_hbm, NM_P // N_TILES)


def _sc_counts(rs2d, rd2d, ones16, zu, zm):
    return pl.kernel(
        _sc_counts_body,
        out_type=(jax.ShapeDtypeStruct((NU_P, 16), f32),
                  jax.ShapeDtypeStruct((NM_P, 16), f32)),
        scratch_types=[
            pltpu.VMEM_SHARED((NU_P, 16), f32),
            pltpu.VMEM((2, 5, 128), i32),
            pltpu.VMEM((128, 16), f32),
            pltpu.SemaphoreType.DMA,
            pltpu.SemaphoreType.DMA,
        ],
        **_mesh(),
    )(rs2d, rd2d, ones16, zu, zm)


def _sc_pass1_body(pqc_hbm, rs2d_hbm, rd2d_hbm, z_hbm, out_hbm,
                   acc_sh, ig_v, is_v, buf_v, semg, sems, semi):
    c = lax.axis_index("core")
    s = lax.axis_index("subcore")
    per = NU_P // N_TILES
    for j in range(4):
        chunk = c * 4 + j
        pltpu.sync_copy(z_hbm, acc_sh.at[pl.ds(s * per, per), :])
        plsc.subcore_barrier()
        _seg_pass(pqc_hbm.at[chunk], rd2d_hbm, rs2d_hbm, acc_sh,
                  ig_v, is_v, buf_v, semg, sems, semi,
                  base_row=s * 800, n_groups=80, K=10, B=40)
        plsc.subcore_barrier()
        pltpu.sync_copy(acc_sh.at[pl.ds(s * per, per), :],
                        out_hbm.at[chunk].at[pl.ds(s * per, per), :])
        plsc.subcore_barrier()


def _sc_pass1(pqc, rs2d, rd2d, zer):
    return pl.kernel(
        _sc_pass1_body,
        out_type=jax.ShapeDtypeStruct((8, NU_P, 32), f32),
        scratch_types=[
            pltpu.VMEM_SHARED((NU_P, 32), f32),
            pltpu.VMEM((2, 10, 40), i32),
            pltpu.VMEM((2, 10, 40), i32),
            pltpu.VMEM((2, 400, 32), f32),
            pltpu.SemaphoreType.DMA,
            pltpu.SemaphoreType.DMA,
            pltpu.SemaphoreType.DMA,
        ],
        **_mesh(),
    )(pqc, rs2d, rd2d, zer)


def _sc_pass2_body(r_hbm, rs2d_hbm, rd2d_hbm, z_hbm, out_hbm,
                   acc_sh, ig_v, is_v, buf_v, semg, sems, semi):
    c = lax.axis_index("core")
    s = lax.axis_index("subcore")
    per = NM_P // N_TILES
    w = c * N_TILES + s
    pltpu.sync_copy(z_hbm, acc_sh.at[pl.ds(s * per, per), :])
    plsc.subcore_barrier()
    _seg_pass(r_hbm, rs2d_hbm, rd2d_hbm, acc_sh, ig_v, is_v, buf_v,
              semg, sems, semi, base_row=w * 500, n_groups=100, K=5, B=32)
    plsc.subcore_barrier()
    pltpu.sync_copy(acc_sh.at[pl.ds(s * per, per), :],
                    out_hbm.at[c].at[pl.ds(s * per, per), :])


def _sc_pass2(r_full, rs2d32, rd2d32, zer):
    return pl.kernel(
        _sc_pass2_body,
        out_type=jax.ShapeDtypeStruct((2, NM_P, H), f32),
        scratch_types=[
            pltpu.VMEM_SHARED((NM_P, H), f32),
            pltpu.VMEM((2, 5, 32), i32),
            pltpu.VMEM((2, 5, 32), i32),
            pltpu.VMEM((2, 160, H), f32),
            pltpu.SemaphoreType.DMA,
            pltpu.SemaphoreType.DMA,
            pltpu.SemaphoreType.DMA,
        ],
        **_mesh(),
    )(r_full, rs2d32, rd2d32, zer)


def _sc_gather_body(u2_hbm, m2_hbm, ls2d_hbm, ld2d_hbm, u_hbm, m_hbm,
                    iu_v, im_v, bu_v, bm_v, semg, semw):
    c = lax.axis_index("core")
    s = lax.axis_index("subcore")
    B = 128
    w = s * 2 + c
    nb = NL_P // (2 * N_TILES * B)

    def wait_wb(par, base):
        pltpu.make_async_copy(bu_v.at[par], u_hbm.at[pl.ds(base, B), :],
                              semw).wait()
        pltpu.make_async_copy(bm_v.at[par], m_hbm.at[pl.ds(base, B), :],
                              semw).wait()

    for b in range(nb):
        par = b % 2
        base = (w * nb + b) * B
        pltpu.sync_copy(ls2d_hbm.at[pl.ds(w * nb + b, 1), :], iu_v.at[par])
        pltpu.sync_copy(ld2d_hbm.at[pl.ds(w * nb + b, 1), :], im_v.at[par])
        if b >= 2:
            wait_wb(par, base)
        d1 = pltpu.async_copy(u2_hbm.at[iu_v.at[par, 0]], bu_v.at[par], semg)
        d2 = pltpu.async_copy(m2_hbm.at[im_v.at[par, 0]], bm_v.at[par], semg)
        d1.wait()
        d2.wait()
        pltpu.async_copy(bu_v.at[par], u_hbm.at[pl.ds(base, B), :], semw)
        pltpu.async_copy(bm_v.at[par], m_hbm.at[pl.ds(base, B), :], semw)
    wait_wb((nb - 2) % 2, 0)
    wait_wb((nb - 1) % 2, 0)


def _sc_gather(u2, m2, ls2d, ld2d):
    return pl.kernel(
        _sc_gather_body,
        out_type=(jax.ShapeDtypeStruct((NL_P, H), f32),
                  jax.ShapeDtypeStruct((NL_P, H), f32)),
        scratch_types=[
            pltpu.VMEM((2, 1, 128), i32),
            pltpu.VMEM((2, 1, 128), i32),
            pltpu.VMEM((2, 128, H), f32),
            pltpu.VMEM((2, 128, H), f32),
            pltpu.SemaphoreType.DMA,
            pltpu.SemaphoreType.DMA,
        ],
        **_mesh(),
    )(u2, m2, ls2d, ld2d)


def _tc_movies_body(x_ref, wm_ref, bm_ref, wr1_ref, br1_ref, wv1_ref,
                    wv2_ref, m1_ref, pqc_ref):
    x = x_ref[...]
    m0 = jnp.dot(x, wm_ref[...], preferred_element_type=f32) + bm_ref[...]
    m1 = jnp.maximum(
        jnp.dot(m0, wr1_ref[...], preferred_element_type=f32) + br1_ref[...],
        0.0)
    p = jnp.dot(m0, wv1_ref[...], preferred_element_type=f32)
    q = jnp.dot(m1, wv2_ref[...], preferred_element_type=f32)
    m1_ref[...] = m1
    pq = jnp.concatenate([p, q], axis=1)
    blk = pq.shape[0]
    pqc_ref[...] = jnp.transpose(pq.reshape(blk, 8, 32), (1, 0, 2))


def _tc_movies(x_p, wm, bm, wr1, br1, wv1, wv2):
    blk = 400
    g = NM // blk
    return pl.pallas_call(
        _tc_movies_body,
        grid=(g,),
        in_specs=[
            pl.BlockSpec((blk, MF), lambda i: (i, 0)),
            pl.BlockSpec((MF, H), lambda i: (0, 0)),
            pl.BlockSpec((1, H), lambda i: (0, 0)),
            pl.BlockSpec((H, H), lambda i: (0, 0)),
            pl.BlockSpec((1, H), lambda i: (0, 0)),
            pl.BlockSpec((H, H), lambda i: (0, 0)),
            pl.BlockSpec((H, H), lambda i: (0, 0)),
        ],
        out_specs=[
            pl.BlockSpec((blk, H), lambda i: (i, 0)),
            pl.BlockSpec((8, blk, 32), lambda i: (0, i, 0)),
        ],
        out_shape=[
            jax.ShapeDtypeStruct((NM_P, H), f32),
            jax.ShapeDtypeStruct((8, NM_P, 32), f32),
        ],
    )(x_p, wm, bm, wr1, br1, wv1, wv2)


def _tc_user_body(spq_ref, cnt_ref, wlr2_ref, wrv2_ref, bv1_ref, bv2_ref,
                  r_ref, u2_ref):
    spq = spq_ref[...]
    inv = 1.0 / jnp.maximum(cnt_ref[...], 1.0)
    b1 = bv1_ref[...]
    b2 = bv2_ref[...]
    u1 = jnp.concatenate(
        [jnp.maximum(spq[j] * inv + b1[:, 32 * j:32 * j + 32], 0.0)
         for j in range(4)], axis=1)
    r_ref[...] = jnp.dot(u1, wlr2_ref[...], preferred_element_type=f32)
    u2_ref[...] = jnp.concatenate(
        [spq[4 + j] * inv + b2[:, 32 * j:32 * j + 32] for j in range(4)],
        axis=1) + jnp.dot(u1, wrv2_ref[...], preferred_element_type=f32)


def _tc_user(spq, cnt_u, wlr2, wrv2, bv1, bv2):
    blk = 512
    g = NU_P // blk
    return pl.pallas_call(
        _tc_user_body,
        grid=(g,),
        in_specs=[
            pl.BlockSpec((8, blk, 32), lambda i: (0, i, 0)),
            pl.BlockSpec((blk, 1), lambda i: (i, 0)),
            pl.BlockSpec((H, H), lambda i: (0, 0)),
            pl.BlockSpec((H, H), lambda i: (0, 0)),
            pl.BlockSpec((1, H), lambda i: (0, 0)),
            pl.BlockSpec((1, H), lambda i: (0, 0)),
        ],
        out_specs=[
            pl.BlockSpec((blk, H), lambda i: (i, 0)),
            pl.BlockSpec((blk, H), lambda i: (i, 0)),
        ],
        out_shape=[
            jax.ShapeDtypeStruct((NU_P, H), f32),
            jax.ShapeDtypeStruct((NU_P, H), f32),
        ],
    )(spq, cnt_u, wlr2, wrv2, bv1, bv2)


def _tc_movie2_body(sr_ref, cnt_ref, m1_ref, wr2_ref, br2_ref, m2_ref):
    sr = sr_ref[...]
    inv = 1.0 / jnp.maximum(cnt_ref[...], 1.0)
    m2_ref[...] = ((sr[0] + sr[1]) * inv + br2_ref[...] +
                   jnp.dot(m1_ref[...], wr2_ref[...],
                           preferred_element_type=f32))


def _tc_movie2(sr, cnt_m, m1, wr2, br2):
    blk = 512
    g = NM_P // blk
    return pl.pallas_call(
        _tc_movie2_body,
        grid=(g,),
        in_specs=[
            pl.BlockSpec((2, blk, H), lambda i: (0, i, 0)),
            pl.BlockSpec((blk, 1), lambda i: (i, 0)),
            pl.BlockSpec((blk, H), lambda i: (i, 0)),
            pl.BlockSpec((H, H), lambda i: (0, 0)),
            pl.BlockSpec((1, H), lambda i: (0, 0)),
        ],
        out_specs=pl.BlockSpec((blk, H), lambda i: (i, 0)),
        out_shape=jax.ShapeDtypeStruct((NM_P, H), f32),
    )(sr, cnt_m, m1, wr2, br2)


def _tc_dot_body(u_ref, m_ref, o_ref):
    o_ref[...] = jnp.sum(u_ref[...] * m_ref[...], axis=1, keepdims=True)


def _tc_dot(u, m):
    blk = 512
    g = NL_P // blk
    return pl.pallas_call(
        _tc_dot_body,
        grid=(g,),
        in_specs=[
            pl.BlockSpec((blk, H), lambda i: (i, 0)),
            pl.BlockSpec((blk, H), lambda i: (i, 0)),
        ],
        out_specs=pl.BlockSpec((blk, 1), lambda i: (i, 0)),
        out_shape=jax.ShapeDtypeStruct((NL_P, 1), f32),
    )(u, m)


def kernel(movie_x, W_movie, b_movie, Wl_r1, bl_r1, Wr_r1, Wl_v1, bl_v1,
           Wr_v1, Wl_r2, bl_r2, Wr_r2, Wl_v2, bl_v2, Wr_v2,
           rates_src, rates_dst, label_src, label_dst):
    pad_u = NU + jnp.arange(NE_P - NE, dtype=i32) % (NU_P - NU)
    pad_m = NM + jnp.arange(NE_P - NE, dtype=i32) % (NM_P - NM)
    rs_p = jnp.concatenate([rates_src.astype(i32), pad_u])
    rd_p = jnp.concatenate([rates_dst.astype(i32), pad_m])
    rs2d = rs_p.reshape(NE_P // 128, 128)
    rd2d = rd_p.reshape(NE_P // 128, 128)
    rs2d64 = rs_p.reshape(NE_P // 64, 64)
    rd2d64 = rd_p.reshape(NE_P // 64, 64)
    rs2d32 = rs_p.reshape(NE_P // 32, 32)
    rd2d32 = rd_p.reshape(NE_P // 32, 32)
    rs2d40 = rs_p.reshape(NE_P // 40, 40)
    rd2d40 = rd_p.reshape(NE_P // 40, 40)
    ls2d = jnp.concatenate([label_src.astype(i32),
                            jnp.zeros((NL_P - NL,), i32)]).reshape(
                                NL_P // 128, 128)
    ld2d = jnp.concatenate([label_dst.astype(i32),
                            jnp.zeros((NL_P - NL,), i32)]).reshape(
                                NL_P // 128, 128)
    bm = b_movie.reshape(1, H)
    br1 = bl_r1.reshape(1, H)
    bv1 = bl_v1.reshape(1, H)
    bv2 = bl_v2.reshape(1, H)
    br2 = bl_r2.reshape(1, H)

    ones16 = jnp.ones((128, 16), f32)
    zu16 = jnp.zeros((NU_P // N_TILES, 16), f32)
    zm16 = jnp.zeros((NM_P // N_TILES, 16), f32)
    z1 = jnp.zeros((NU_P // N_TILES, 32), f32)
    z2 = jnp.zeros((NM_P // N_TILES, H), f32)

    movie1, pqc = _tc_movies(movie_x, W_movie, bm, Wr_r1, br1, Wl_v1, Wl_v2)

    cu16, cm16 = _sc_counts(rs2d, rd2d, ones16, zu16, zm16)
    cnt_u = cu16[:, :1]
    cnt_m = cm16[:, :1]

    spq = _sc_pass1(pqc, rs2d40, rd2d40, z1)
    r_full, user2 = _tc_user(spq, cnt_u, Wl_r2, Wr_v2, bv1, bv2)
    sr = _sc_pass2(r_full, rs2d32, rd2d32, z2)
    movie2 = _tc_movie2(sr, cnt_m, movie1, Wr_r2, br2)
    u_rows, m_rows = _sc_gather(user2, movie2, ls2d, ld2d)
    out = _tc_dot(u_rows, m_rows)
    return out.reshape(NL_P)[:NL]
